# baseline (speedup 1.0000x reference)
import functools

import jax
import jax.numpy as jnp
from jax import lax
from jax.experimental import pallas as pl
from jax.experimental.pallas import tpu as pltpu
from jax.experimental.pallas import tpu_sc as plsc

D = 128
DEG = 32
N = 10000
NPAD = 10240
L = 16
NC, NS = 2, 16
NW = NC * NS
G = NPAD // L
GPW = G // NW
NCHUNK = 4
JC = DEG // NCHUNK
CC = 8
PBLK = 256
BLK = 128


def _proj_body(feat_ref, w_ref, te_ref, h_ref, tp_ref):
    h = jnp.dot(feat_ref[...], w_ref[...].T, preferred_element_type=jnp.float32)
    h_ref[...] = h
    tp_ref[...] = jnp.dot(h, te_ref[...].T, preferred_element_type=jnp.float32)


def _proj(feat, w, te32):
    n = feat.shape[0]
    return pl.pallas_call(
        _proj_body,
        grid=(n // PBLK,),
        in_specs=[
            pl.BlockSpec((PBLK, D), lambda i: (i, 0)),
            pl.BlockSpec((D, D), lambda i: (0, 0)),
            pl.BlockSpec((DEG, D), lambda i: (0, 0)),
        ],
        out_specs=[
            pl.BlockSpec((PBLK, D), lambda i: (i, 0)),
            pl.BlockSpec((PBLK, DEG), lambda i: (i, 0)),
        ],
        out_shape=[
            jax.ShapeDtypeStruct((n, D), jnp.float32),
            jax.ShapeDtypeStruct((n, DEG), jnp.float32),
        ],
    )(feat, w, te32)


def _rank_body(t_ref, ro_ref, last_ref):
    t = t_ref[...]
    tj = t[:, :, None]
    tk = t[:, None, :]
    kk = lax.broadcasted_iota(jnp.int32, (1, DEG, DEG), 2)
    jj = lax.broadcasted_iota(jnp.int32, (1, DEG, DEG), 1)
    gt = (tk > tj) | ((tk == tj) & (kk > jj))
    ro_ref[...] = jnp.sum(gt.astype(jnp.int32), axis=2)
    ge_first = (tk > tj) | ((tk == tj) & (kk < jj))
    fm = (jnp.sum(ge_first.astype(jnp.int32), axis=2) == 0)
    jvec = lax.broadcasted_iota(jnp.int32, (1, DEG), 1)
    last = jnp.sum(jnp.where(fm, jvec, 0), axis=1, keepdims=True)
    last_ref[...] = jnp.broadcast_to(last, last_ref.shape)


def _rank(t):
    n = t.shape[0]
    return pl.pallas_call(
        _rank_body,
        grid=(n // BLK,),
        in_specs=[pl.BlockSpec((BLK, DEG), lambda i: (i, 0))],
        out_specs=[
            pl.BlockSpec((BLK, DEG), lambda i: (i, 0)),
            pl.BlockSpec((BLK, L), lambda i: (i, 0)),
        ],
        out_shape=[
            jax.ShapeDtypeStruct((n, DEG), jnp.int32),
            jax.ShapeDtypeStruct((n, L), jnp.int32),
        ],
    )(t)


def _gate_body(hlm_ref, beta_ref, hs_ref, feat_ref, tek_ref, wg_ref, out_ref):
    h_long = hlm_ref[...] + jnp.dot(beta_ref[...], tek_ref[...],
                                    preferred_element_type=jnp.float32)
    msg = jnp.concatenate([h_long, hs_ref[...]], axis=1)
    new = lax.dot_general(msg, wg_ref[...], (((1,), (1,)), ((), ())),
                          preferred_element_type=jnp.float32)
    x = new + feat_ref[...]
    out_ref[...] = jnp.where(x > 0, x, jnp.exp(x) - 1.0)


def _gate(hlm, beta, hs, feat, tek32, wg):
    n = hlm.shape[0]
    return pl.pallas_call(
        _gate_body,
        grid=(n // PBLK,),
        in_specs=[
            pl.BlockSpec((PBLK, D), lambda i: (i, 0)),
            pl.BlockSpec((PBLK, DEG), lambda i: (i, 0)),
            pl.BlockSpec((PBLK, D), lambda i: (i, 0)),
            pl.BlockSpec((PBLK, D), lambda i: (i, 0)),
            pl.BlockSpec((DEG, D), lambda i: (0, 0)),
            pl.BlockSpec((D, 2 * D), lambda i: (0, 0)),
        ],
        out_specs=pl.BlockSpec((PBLK, D), lambda i: (i, 0)),
        out_shape=jax.ShapeDtypeStruct((n, D), jnp.float32),
    )(hlm, beta, hs, feat, tek32, wg)


TP_OFF, H_OFF, AUXF = 0, DEG, DEG + D
RO_OFF, LAST_OFF, AUXI = 0, DEG, DEG + 1
HS_OFF, BETA_OFF, OUTR = D, 2 * D, 2 * D + DEG


def _sc_attention(table, idx_r, auxf_r, auxi_r):
    mesh = plsc.VectorSubcoreMesh(core_axis_name="c", subcore_axis_name="s",
                                  num_cores=NC, num_subcores=NS)
    f32 = jnp.float32

    def body(table, idxh, auxfh, auxih, outh,
             idx_v, stage_a, stage_b, auxf_v, auxi_v,
             e_v, alpha_v, e1_v, alpha1_v, out_v, mt,
             sem_a, sem_b):
        wid = lax.axis_index("s") * NC + lax.axis_index("c")
        iota16 = lax.broadcasted_iota(jnp.int32, (L,), 0)
        inv_scale = 1.0 / float(D) ** 0.5
        neg = jnp.full((L,), -3.0e38, jnp.float32)
        zeros = jnp.zeros((L,), jnp.float32)

        def group(gi, _):
            g = wid * GPW + gi
            pltpu.sync_copy(idxh.at[g], idx_v)
            pltpu.sync_copy(auxfh.at[g], auxf_v)
            pltpu.sync_copy(auxih.at[g], auxi_v)
            stages = [stage_a, stage_b]
            sems = [sem_a, sem_b]
            cps = [pltpu.async_copy(table.at[idx_v.at[0]], stage_a, sem_a)]

            for jc in range(NCHUNK):
                if jc + 1 < NCHUNK:
                    cps.append(pltpu.async_copy(table.at[idx_v.at[jc + 1]],
                                                stages[(jc + 1) % 2],
                                                sems[(jc + 1) % 2]))
                cps[jc].wait()
                buf = stages[jc % 2]

                def cbody(c, eacc, jc=jc, buf=buf):
                    hvec = auxf_v[H_OFF + c]
                    cvec = jnp.full((L,), c, jnp.int32)
                    new = []
                    for j in range(JC):
                        m = plsc.load_gather(buf, [iota16 + (j * L), cvec])
                        mt[jc * JC + j, c] = m
                        new.append(eacc[j] + m * hvec)
                    return tuple(new)

                eacc = lax.fori_loop(0, D, cbody, (zeros,) * JC)
                for j in range(JC):
                    e_v[jc * JC + j] = eacc[j]

            def ebody(j, mx):
                ro_j = auxi_v[RO_OFF + j]
                tpg = plsc.load_gather(auxf_v, [ro_j, iota16])
                ej = (e_v[j] + tpg) * inv_scale
                e_v[j] = ej
                return jnp.maximum(mx, ej)

            mx = lax.fori_loop(0, DEG, ebody, neg)

            def pbody(j, s):
                p = jnp.exp(e_v[j] - mx)
                e_v[j] = p
                return s + p

            rs = 1.0 / lax.fori_loop(0, DEG, pbody, zeros)

            def abody(j, _):
                a = e_v[j] * rs
                alpha_v[j] = a
                ro_j = auxi_v[RO_OFF + j]
                plsc.store_scatter(out_v, [ro_j + BETA_OFF, iota16], a)
                return 0

            lax.fori_loop(0, DEG, abody, 0)

            lastv = auxi_v[LAST_OFF]

            def e1cc(cc, _):
                base = cc * CC
                lems = [plsc.load_gather(
                    mt, [lastv, jnp.full((L,), base + i, jnp.int32), iota16])
                    for i in range(CC)]

                def jbody(j, _):
                    acc = e1_v[j]
                    for i in range(CC):
                        acc = acc + mt[j, base + i] * lems[i]
                    e1_v[j] = acc
                    return 0

                lax.fori_loop(0, DEG, jbody, 0)
                return 0

            def e1z(j, _):
                e1_v[j] = zeros
                return 0

            lax.fori_loop(0, DEG, e1z, 0)
            lax.fori_loop(0, D // CC, e1cc, 0)

            def e1mx(j, mx):
                return jnp.maximum(mx, e1_v[j] * inv_scale)

            mx1 = lax.fori_loop(0, DEG, e1mx, neg)

            def p1body(j, s):
                p = jnp.exp(e1_v[j] * inv_scale - mx1)
                e1_v[j] = p
                return s + p

            rs1 = 1.0 / lax.fori_loop(0, DEG, p1body, zeros)

            def a1body(j, _):
                alpha1_v[j] = e1_v[j] * rs1
                return 0

            lax.fori_loop(0, DEG, a1body, 0)

            def hcc(cc, _):
                base = cc * CC

                def jbody2(j, carry):
                    a = alpha_v[j]
                    a1 = alpha1_v[j]
                    outl = []
                    outs = []
                    for i in range(CC):
                        m = mt[j, base + i]
                        outl.append(carry[i] + a * m)
                        outs.append(carry[CC + i] + a1 * m)
                    return tuple(outl + outs)

                res = lax.fori_loop(0, DEG, jbody2, (zeros,) * (2 * CC))
                for i in range(CC):
                    out_v[base + i] = res[i]
                    out_v[HS_OFF + base + i] = res[CC + i]
                return 0

            lax.fori_loop(0, D // CC, hcc, 0)

            pltpu.sync_copy(out_v, outh.at[g])
            return 0

        lax.fori_loop(0, GPW, group, 0)

    return pl.kernel(
        body,
        compiler_params=pltpu.CompilerParams(needs_layout_passes=False,
                                             use_tc_tiling_on_sc=False),
        out_type=jax.ShapeDtypeStruct((G, OUTR, L), f32),
        mesh=mesh,
        scratch_types=[
            pltpu.VMEM((NCHUNK, JC * L), jnp.int32),
            pltpu.VMEM((JC * L, D), f32),
            pltpu.VMEM((JC * L, D), f32),
            pltpu.VMEM((AUXF, L), f32),
            pltpu.VMEM((AUXI, L), jnp.int32),
            pltpu.VMEM((DEG, L), f32),
            pltpu.VMEM((DEG, L), f32),
            pltpu.VMEM((DEG, L), f32),
            pltpu.VMEM((DEG, L), f32),
            pltpu.VMEM((OUTR, L), f32),
            pltpu.VMEM((DEG, D, L), f32),
            pltpu.SemaphoreType.DMA,
            pltpu.SemaphoreType.DMA,
        ],
    )(table, idx_r, auxf_r, auxi_r)


def _pad_rows(x, npad):
    return jnp.pad(x, ((0, npad - x.shape[0]),) + ((0, 0),) * (x.ndim - 1))


def _to_groups(x):
    return x.reshape(G, L, x.shape[1]).transpose(0, 2, 1)


def _from_groups(x):
    return x.transpose(0, 2, 1).reshape(NPAD, x.shape[1])


def kernel(user_feat, item_feat, Wu, Wi, Wg_u, Wg_i, i_te, i_te_k, u_te,
           u_te_k, item_neighbors, item_nbr_time, user_neighbors,
           user_nbr_time):
    user_h, tp_u = _proj(_pad_rows(user_feat, NPAD), Wu, u_te[:DEG])
    item_h, tp_i = _proj(_pad_rows(item_feat, NPAD), Wi, i_te[:DEG])

    def side(src_h, nbrs, times, tp, dst_h, dst_feat, tek, wg):
        nbrs = _pad_rows(nbrs.astype(jnp.int32), NPAD)
        t = _pad_rows(times.astype(jnp.int32), NPAD)
        ro, last = _rank(t)
        idx_r = (nbrs.reshape(G, L, NCHUNK, JC)
                 .transpose(0, 2, 3, 1).reshape(G, NCHUNK, JC * L))
        last_r = last[:, 0].reshape(G, 1, L)
        auxf_r = jnp.concatenate([_to_groups(tp), _to_groups(dst_h)], axis=1)
        auxi_r = jnp.concatenate([_to_groups(ro), last_r], axis=1)
        out_r = _sc_attention(src_h, idx_r, auxf_r, auxi_r)
        out = _gate(_from_groups(out_r[:, :D]),
                    _from_groups(out_r[:, BETA_OFF:]),
                    _from_groups(out_r[:, D:BETA_OFF]),
                    _pad_rows(dst_feat, NPAD),
                    tek[:DEG], wg)
        return out[:N]

    item_out = side(user_h, item_neighbors, item_nbr_time, tp_i, item_h,
                    item_feat, i_te_k, Wg_i)
    user_out = side(item_h, user_neighbors, user_nbr_time, tp_u, user_h,
                    user_feat, u_te_k, Wg_u)
    return user_out, item_out


# speedup vs baseline: 1.3612x; 1.3612x over previous
import functools

import jax
import jax.numpy as jnp
from jax import lax
from jax.experimental import pallas as pl
from jax.experimental.pallas import tpu as pltpu
from jax.experimental.pallas import tpu_sc as plsc

D = 128
DEG = 32
N = 10000
NPAD = 10240
L = 16
NC, NS = 2, 16
NW = NC * NS
G = NPAD // L
GPW = G // NW
NCHUNK = 4
JC = DEG // NCHUNK
CC = 8
PBLK = 256
BLK = 128


def _proj_body(feat_ref, w_ref, te_ref, h_ref, tp_ref):
    h = jnp.dot(feat_ref[...], w_ref[...].T, preferred_element_type=jnp.float32)
    h_ref[...] = h
    tp_ref[...] = jnp.dot(h, te_ref[...].T, preferred_element_type=jnp.float32)


def _proj(feat, w, te32):
    n = feat.shape[0]
    return pl.pallas_call(
        _proj_body,
        grid=(n // PBLK,),
        in_specs=[
            pl.BlockSpec((PBLK, D), lambda i: (i, 0)),
            pl.BlockSpec((D, D), lambda i: (0, 0)),
            pl.BlockSpec((DEG, D), lambda i: (0, 0)),
        ],
        out_specs=[
            pl.BlockSpec((PBLK, D), lambda i: (i, 0)),
            pl.BlockSpec((PBLK, DEG), lambda i: (i, 0)),
        ],
        out_shape=[
            jax.ShapeDtypeStruct((n, D), jnp.float32),
            jax.ShapeDtypeStruct((n, DEG), jnp.float32),
        ],
    )(feat, w, te32)


def _rank_body(t_ref, ro_ref, last_ref):
    t = t_ref[...]
    tj = t[:, :, None]
    tk = t[:, None, :]
    kk = lax.broadcasted_iota(jnp.int32, (1, DEG, DEG), 2)
    jj = lax.broadcasted_iota(jnp.int32, (1, DEG, DEG), 1)
    gt = (tk > tj) | ((tk == tj) & (kk > jj))
    ro_ref[...] = jnp.sum(gt.astype(jnp.int32), axis=2)
    ge_first = (tk > tj) | ((tk == tj) & (kk < jj))
    fm = (jnp.sum(ge_first.astype(jnp.int32), axis=2) == 0)
    jvec = lax.broadcasted_iota(jnp.int32, (1, DEG), 1)
    last = jnp.sum(jnp.where(fm, jvec, 0), axis=1, keepdims=True)
    last_ref[...] = jnp.broadcast_to(last, last_ref.shape)


def _rank(t):
    n = t.shape[0]
    return pl.pallas_call(
        _rank_body,
        grid=(n // BLK,),
        in_specs=[pl.BlockSpec((BLK, DEG), lambda i: (i, 0))],
        out_specs=[
            pl.BlockSpec((BLK, DEG), lambda i: (i, 0)),
            pl.BlockSpec((BLK, L), lambda i: (i, 0)),
        ],
        out_shape=[
            jax.ShapeDtypeStruct((n, DEG), jnp.int32),
            jax.ShapeDtypeStruct((n, L), jnp.int32),
        ],
    )(t)


def _gate_body(hlm_ref, beta_ref, hs_ref, feat_ref, tek_ref, wg_ref, out_ref):
    h_long = hlm_ref[...] + jnp.dot(beta_ref[...], tek_ref[...],
                                    preferred_element_type=jnp.float32)
    msg = jnp.concatenate([h_long, hs_ref[...]], axis=1)
    new = lax.dot_general(msg, wg_ref[...], (((1,), (1,)), ((), ())),
                          preferred_element_type=jnp.float32)
    x = new + feat_ref[...]
    out_ref[...] = jnp.where(x > 0, x, jnp.exp(x) - 1.0)


def _gate(hlm, beta, hs, feat, tek32, wg):
    n = hlm.shape[0]
    return pl.pallas_call(
        _gate_body,
        grid=(n // PBLK,),
        in_specs=[
            pl.BlockSpec((PBLK, D), lambda i: (i, 0)),
            pl.BlockSpec((PBLK, DEG), lambda i: (i, 0)),
            pl.BlockSpec((PBLK, D), lambda i: (i, 0)),
            pl.BlockSpec((PBLK, D), lambda i: (i, 0)),
            pl.BlockSpec((DEG, D), lambda i: (0, 0)),
            pl.BlockSpec((D, 2 * D), lambda i: (0, 0)),
        ],
        out_specs=pl.BlockSpec((PBLK, D), lambda i: (i, 0)),
        out_shape=jax.ShapeDtypeStruct((n, D), jnp.float32),
    )(hlm, beta, hs, feat, tek32, wg)


TP_OFF, H_OFF, AUXF = 0, DEG, DEG + D
RO_OFF, LAST_OFF, AUXI = 0, DEG, DEG + 1
HS_OFF, BETA_OFF, OUTR = D, 2 * D, 2 * D + DEG


def _sc_attention(table, idx_r, auxf_r, auxi_r):
    mesh = plsc.VectorSubcoreMesh(core_axis_name="c", subcore_axis_name="s",
                                  num_cores=NC, num_subcores=NS)
    f32 = jnp.float32

    def body(table, idxh, auxfh, auxih, outh,
             idx_v, stage_a, stage_b, auxf_v, auxi_v,
             e_v, alpha_v, e1_v, alpha1_v, out_v, mt,
             sem_a, sem_b):
        wid = lax.axis_index("s") * NC + lax.axis_index("c")
        iota16 = lax.broadcasted_iota(jnp.int32, (L,), 0)
        inv_scale = 1.0 / float(D) ** 0.5
        neg = jnp.full((L,), -3.0e38, jnp.float32)
        zeros = jnp.zeros((L,), jnp.float32)

        def group(gi, _):
            g = wid * GPW + gi
            pltpu.sync_copy(idxh.at[g], idx_v)
            pltpu.sync_copy(auxfh.at[g], auxf_v)
            pltpu.sync_copy(auxih.at[g], auxi_v)
            stages = [stage_a, stage_b]
            sems = [sem_a, sem_b]
            cps = [pltpu.async_copy(table.at[idx_v.at[0]], stage_a, sem_a)]

            for jc in range(NCHUNK):
                if jc + 1 < NCHUNK:
                    cps.append(pltpu.async_copy(table.at[idx_v.at[jc + 1]],
                                                stages[(jc + 1) % 2],
                                                sems[(jc + 1) % 2]))
                cps[jc].wait()
                buf = stages[jc % 2]

                @plsc.parallel_loop(0, D, unroll=4, carry=(zeros,) * JC)
                def cbody(c, eacc, jc=jc, buf=buf):
                    hvec = auxf_v[H_OFF + c]
                    cvec = jnp.full((L,), c, jnp.int32)
                    new = []
                    for j in range(JC):
                        m = plsc.load_gather(buf, [iota16 + (j * L), cvec])
                        mt[jc * JC + j, c] = m
                        new.append(eacc[j] + m * hvec)
                    return tuple(new)

                for j in range(JC):
                    e_v[jc * JC + j] = cbody[j]

            def ebody(j, mx):
                ro_j = auxi_v[RO_OFF + j]
                tpg = plsc.load_gather(auxf_v, [ro_j, iota16])
                ej = (e_v[j] + tpg) * inv_scale
                e_v[j] = ej
                return jnp.maximum(mx, ej)

            mx = lax.fori_loop(0, DEG, ebody, neg)

            def pbody(j, s):
                p = jnp.exp(e_v[j] - mx)
                e_v[j] = p
                return s + p

            rs = 1.0 / lax.fori_loop(0, DEG, pbody, zeros)

            def abody(j, _):
                a = e_v[j] * rs
                alpha_v[j] = a
                ro_j = auxi_v[RO_OFF + j]
                plsc.store_scatter(out_v, [ro_j + BETA_OFF, iota16], a)
                return 0

            lax.fori_loop(0, DEG, abody, 0)

            lastv = auxi_v[LAST_OFF]

            def e1cc(cc, _):
                base = cc * CC
                lems = [plsc.load_gather(
                    mt, [lastv, jnp.full((L,), base + i, jnp.int32), iota16])
                    for i in range(CC)]

                @plsc.parallel_loop(0, DEG, unroll=2)
                def jbody(j):
                    acc = e1_v[j]
                    for i in range(CC):
                        acc = acc + mt[j, base + i] * lems[i]
                    e1_v[j] = acc

                return 0

            def e1z(j, _):
                e1_v[j] = zeros
                return 0

            lax.fori_loop(0, DEG, e1z, 0)
            lax.fori_loop(0, D // CC, e1cc, 0)

            def e1mx(j, mx):
                return jnp.maximum(mx, e1_v[j] * inv_scale)

            mx1 = lax.fori_loop(0, DEG, e1mx, neg)

            def p1body(j, s):
                p = jnp.exp(e1_v[j] * inv_scale - mx1)
                e1_v[j] = p
                return s + p

            rs1 = 1.0 / lax.fori_loop(0, DEG, p1body, zeros)

            def a1body(j, _):
                alpha1_v[j] = e1_v[j] * rs1
                return 0

            lax.fori_loop(0, DEG, a1body, 0)

            def hcc(cc, _):
                base = cc * CC

                @plsc.parallel_loop(0, DEG, unroll=2,
                                    carry=(zeros,) * (2 * CC))
                def jbody2(j, carry):
                    a = alpha_v[j]
                    a1 = alpha1_v[j]
                    outl = []
                    outs = []
                    for i in range(CC):
                        m = mt[j, base + i]
                        outl.append(carry[i] + a * m)
                        outs.append(carry[CC + i] + a1 * m)
                    return tuple(outl + outs)

                for i in range(CC):
                    out_v[base + i] = jbody2[i]
                    out_v[HS_OFF + base + i] = jbody2[CC + i]
                return 0

            lax.fori_loop(0, D // CC, hcc, 0)

            pltpu.sync_copy(out_v, outh.at[g])
            return 0

        lax.fori_loop(0, GPW, group, 0)

    return pl.kernel(
        body,
        compiler_params=pltpu.CompilerParams(needs_layout_passes=False,
                                             use_tc_tiling_on_sc=False),
        out_type=jax.ShapeDtypeStruct((G, OUTR, L), f32),
        mesh=mesh,
        scratch_types=[
            pltpu.VMEM((NCHUNK, JC * L), jnp.int32),
            pltpu.VMEM((JC * L, D), f32),
            pltpu.VMEM((JC * L, D), f32),
            pltpu.VMEM((AUXF, L), f32),
            pltpu.VMEM((AUXI, L), jnp.int32),
            pltpu.VMEM((DEG, L), f32),
            pltpu.VMEM((DEG, L), f32),
            pltpu.VMEM((DEG, L), f32),
            pltpu.VMEM((DEG, L), f32),
            pltpu.VMEM((OUTR, L), f32),
            pltpu.VMEM((DEG, D, L), f32),
            pltpu.SemaphoreType.DMA,
            pltpu.SemaphoreType.DMA,
        ],
    )(table, idx_r, auxf_r, auxi_r)


def _pad_rows(x, npad):
    return jnp.pad(x, ((0, npad - x.shape[0]),) + ((0, 0),) * (x.ndim - 1))


def _to_groups(x):
    return x.reshape(G, L, x.shape[1]).transpose(0, 2, 1)


def _from_groups(x):
    return x.transpose(0, 2, 1).reshape(NPAD, x.shape[1])


def kernel(user_feat, item_feat, Wu, Wi, Wg_u, Wg_i, i_te, i_te_k, u_te,
           u_te_k, item_neighbors, item_nbr_time, user_neighbors,
           user_nbr_time):
    user_h, tp_u = _proj(_pad_rows(user_feat, NPAD), Wu, u_te[:DEG])
    item_h, tp_i = _proj(_pad_rows(item_feat, NPAD), Wi, i_te[:DEG])

    def side(src_h, nbrs, times, tp, dst_h, dst_feat, tek, wg):
        nbrs = _pad_rows(nbrs.astype(jnp.int32), NPAD)
        t = _pad_rows(times.astype(jnp.int32), NPAD)
        ro, last = _rank(t)
        idx_r = (nbrs.reshape(G, L, NCHUNK, JC)
                 .transpose(0, 2, 3, 1).reshape(G, NCHUNK, JC * L))
        last_r = last[:, 0].reshape(G, 1, L)
        auxf_r = jnp.concatenate([_to_groups(tp), _to_groups(dst_h)], axis=1)
        auxi_r = jnp.concatenate([_to_groups(ro), last_r], axis=1)
        out_r = _sc_attention(src_h, idx_r, auxf_r, auxi_r)
        out = _gate(_from_groups(out_r[:, :D]),
                    _from_groups(out_r[:, BETA_OFF:]),
                    _from_groups(out_r[:, D:BETA_OFF]),
                    _pad_rows(dst_feat, NPAD),
                    tek[:DEG], wg)
        return out[:N]

    item_out = side(user_h, item_neighbors, item_nbr_time, tp_i, item_h,
                    item_feat, i_te_k, Wg_i)
    user_out = side(item_h, user_neighbors, user_nbr_time, tp_u, user_h,
                    user_feat, u_te_k, Wg_u)
    return user_out, item_out
